# baseline (speedup 1.0000x reference)
import functools

import numpy as np

import jax
import jax.numpy as jnp
from jax import lax
from jax.experimental import pallas as pl
from jax.experimental.pallas import tpu as pltpu
from jax.experimental.pallas import tpu_sc as plsc

_L = 200
_D = 64
_NSLOT = 6
_TPS = 5
_NI = _L // _TPS
_B = 4
_NB = _NI // _B
_V = 1000000


def _extract(iota, iv, ln, zero16):
    return jnp.max(jnp.where(iota == jnp.int32(ln), iv, zero16))


def _sc_body(i0, i1, i2, i3, i4, i5, rows16, t0, t1, t2, t3, t4, t5, out,
             idx_v, rows_v, part_v, outv_v, widx_v, shared, sem0, sem1):
    c = lax.axis_index("c")
    si = lax.axis_index("s")
    idxs = (i0, i1, i2, i3, i4, i5)
    tabs = (t0, t1, t2, t3, t4, t5)
    sems = (sem0, sem1)
    slot_local = si // _TPS
    t = si % _TPS
    is_worker = si < 15
    iota = lax.iota(jnp.int32, 16)
    zero16 = jnp.zeros((16,), jnp.int32)
    base = _NI * t

    @pl.when(si == 15)
    def _():
        zf = jnp.zeros((16,), jnp.float32)
        for r in range(16):
            for cc in range(4):
                part_v[r, pl.ds(cc * 16, 16)] = zf
        pltpu.sync_copy(part_v, shared)

    plsc.subcore_barrier()

    for k in range(_NSLOT):
        @pl.when(jnp.logical_and(is_worker, (c * 3 + slot_local) == k))
        def _(k=k):
            pltpu.sync_copy(idxs[k], idx_v.at[pl.ds(0, _L)])

    def fire(w):
        for k in range(_NSLOT):
            @pl.when(jnp.logical_and(is_worker, (c * 3 + slot_local) == k))
            def _(k=k, w=w):
                for j in range(_B):
                    n = w * _B + j
                    g, ln = n // 16, n % 16
                    iv = idx_v[pl.ds(base + g * 16, 16)]
                    s = _extract(iota, iv, ln, zero16)
                    s128 = pl.multiple_of((s >> 7) << 7, 128)
                    pltpu.async_copy(
                        tabs[k].at[:, :, pl.ds(s128, 128)],
                        rows_v.at[w % 2, j], sems[w % 2])

    avecs = [(iota + jnp.int32(cc * 16)) >> 3 for cc in range(4)]
    bvecs = [(iota + jnp.int32(cc * 16)) & jnp.int32(7) for cc in range(4)]

    def drain_acc(w, accs):
        @pl.when(is_worker)
        def _():
            for j in range(_B):
                pltpu.make_async_copy(
                    tabs[0].at[:, :, pl.ds(0, 128)],
                    rows_v.at[w % 2, j], sems[w % 2]).wait()
        new = []
        for cc in range(4):
            acc = accs[cc]
            for j in range(_B):
                n = w * _B + j
                g, ln = n // 16, n % 16
                iv = idx_v[pl.ds(base + g * 16, 16)]
                rm = _extract(iota, iv & jnp.int32(127), ln, zero16)
                rv = jnp.full((16,), rm, jnp.int32)
                jv = jnp.full((16,), j, jnp.int32)
                bufv = jnp.full((16,), w % 2, jnp.int32)
                vals = plsc.load_gather(
                    rows_v, [bufv, jv, avecs[cc], bvecs[cc], rv])
                acc = jnp.where(is_worker, acc + vals, acc)
            new.append(acc)
        return new

    accs = [jnp.zeros((16,), jnp.float32) for _ in range(4)]
    fire(0)
    for w in range(1, _NB):
        fire(w)
        accs = drain_acc(w - 1, accs)
    accs = drain_acc(_NB - 1, accs)

    @pl.when(is_worker)
    def _():
        zf = jnp.zeros((16,), jnp.float32)
        for r in range(16):
            for cc in range(4):
                part_v[r, pl.ds(cc * 16, 16)] = zf
        for kk in range(3):
            @pl.when(slot_local == kk)
            def _(kk=kk):
                for cc in range(4):
                    part_v[kk, pl.ds(cc * 16, 16)] = accs[cc]
        pltpu.sync_copy(rows16, widx_v)
        pltpu.sync_copy(part_v, shared.at[widx_v], add=True)

    plsc.subcore_barrier()

    @pl.when(si == 15)
    def _():
        pltpu.sync_copy(shared, part_v)
        scale = jnp.float32(1.0 / _L)
        for kk in range(3):
            for cc in range(4):
                outv_v[pl.ds(kk * 64 + cc * 16, 16)] = (
                    part_v[kk, pl.ds(cc * 16, 16)] * scale)
        for cj in range(2):
            @pl.when(c == cj)
            def _(cj=cj):
                pltpu.sync_copy(outv_v, out.at[pl.ds(cj * 192, 192)])


@jax.jit
def _run(idx_list, tab_list):
    kfn = pl.kernel(
        _sc_body,
        out_type=jax.ShapeDtypeStruct((_NSLOT * _D,), jnp.float32),
        mesh=plsc.VectorSubcoreMesh(core_axis_name="c", subcore_axis_name="s"),
        compiler_params=pltpu.CompilerParams(needs_layout_passes=False),
        scratch_types=[
            pltpu.VMEM((256,), jnp.int32),
            pltpu.VMEM((2, _B, 8, 8, 128), jnp.float32),
            pltpu.VMEM((16, _D), jnp.float32),
            pltpu.VMEM((192,), jnp.float32),
            pltpu.VMEM((16,), jnp.int32),
            pltpu.VMEM_SHARED((16, _D), jnp.float32),
            pltpu.SemaphoreType.DMA,
            pltpu.SemaphoreType.DMA,
        ],
    )
    rows16 = jnp.asarray(np.arange(16, dtype=np.int32))
    return kfn(*idx_list, rows16, *tab_list)


def kernel(scene, subject, action, object, purpose, result,
           W_scene, W_subject, W_action, W_object, W_purpose, W_result):
    idx_list = [x.reshape(_L).astype(jnp.int32)
                for x in (scene, subject, action, object, purpose, result)]
    tab_list = [w.T.reshape(8, 8, _V)
                for w in (W_scene, W_subject, W_action, W_object,
                          W_purpose, W_result)]
    return _run(idx_list, tab_list)


# speedup vs baseline: 1.0642x; 1.0642x over previous
import functools

import numpy as np

import jax
import jax.numpy as jnp
from jax import lax
from jax.experimental import pallas as pl
from jax.experimental.pallas import tpu as pltpu
from jax.experimental.pallas import tpu_sc as plsc

_L = 200
_D = 64
_NSLOT = 6
_TPS = 5
_NI = _L // _TPS
_B = 5
_NB = _NI // _B
_V = 1000000


def _extract(iota, iv, ln, zero16):
    return jnp.max(jnp.where(iota == jnp.int32(ln), iv, zero16))


def _sc_body(i0, i1, i2, i3, i4, i5, rows16, t0, t1, t2, t3, t4, t5, out,
             idx_v, rows_v, part_v, outv_v, widx_v, shared, sem0, sem1, sem2):
    c = lax.axis_index("c")
    si = lax.axis_index("s")
    idxs = (i0, i1, i2, i3, i4, i5)
    tabs = (t0, t1, t2, t3, t4, t5)
    sems = (sem0, sem1, sem2)
    slot_local = si // _TPS
    t = si % _TPS
    is_worker = si < 15
    iota = lax.iota(jnp.int32, 16)
    zero16 = jnp.zeros((16,), jnp.int32)
    base = _NI * t

    @pl.when(si == 15)
    def _():
        zf = jnp.zeros((16,), jnp.float32)
        for r in range(16):
            for cc in range(4):
                part_v[r, pl.ds(cc * 16, 16)] = zf
        pltpu.sync_copy(part_v, shared)

    plsc.subcore_barrier()

    for k in range(_NSLOT):
        @pl.when(jnp.logical_and(is_worker, (c * 3 + slot_local) == k))
        def _(k=k):
            pltpu.sync_copy(idxs[k], idx_v.at[pl.ds(0, _L)])

    def fire(w):
        for k in range(_NSLOT):
            @pl.when(jnp.logical_and(is_worker, (c * 3 + slot_local) == k))
            def _(k=k, w=w):
                for j in range(_B):
                    n = w * _B + j
                    g, ln = n // 16, n % 16
                    iv = idx_v[pl.ds(base + g * 16, 16)]
                    s = _extract(iota, iv, ln, zero16)
                    s128 = pl.multiple_of((s >> 7) << 7, 128)
                    pltpu.async_copy(
                        tabs[k].at[:, :, pl.ds(s128, 128)],
                        rows_v.at[w % 3, j], sems[w % 3])

    avecs = [(iota + jnp.int32(cc * 16)) >> 3 for cc in range(4)]
    bvecs = [(iota + jnp.int32(cc * 16)) & jnp.int32(7) for cc in range(4)]

    def drain_acc(w, accs):
        @pl.when(is_worker)
        def _():
            for j in range(_B):
                pltpu.make_async_copy(
                    tabs[0].at[:, :, pl.ds(0, 128)],
                    rows_v.at[w % 3, j], sems[w % 3]).wait()
        new = []
        for cc in range(4):
            acc = accs[cc]
            for j in range(_B):
                n = w * _B + j
                g, ln = n // 16, n % 16
                iv = idx_v[pl.ds(base + g * 16, 16)]
                rm = _extract(iota, iv & jnp.int32(127), ln, zero16)
                rv = jnp.full((16,), rm, jnp.int32)
                jv = jnp.full((16,), j, jnp.int32)
                bufv = jnp.full((16,), w % 3, jnp.int32)
                vals = plsc.load_gather(
                    rows_v, [bufv, jv, avecs[cc], bvecs[cc], rv])
                acc = jnp.where(is_worker, acc + vals, acc)
            new.append(acc)
        return new

    accs = [jnp.zeros((16,), jnp.float32) for _ in range(4)]
    fire(0)
    fire(1)
    for w in range(2, _NB):
        fire(w)
        accs = drain_acc(w - 2, accs)
    accs = drain_acc(_NB - 2, accs)
    accs = drain_acc(_NB - 1, accs)

    @pl.when(is_worker)
    def _():
        zf = jnp.zeros((16,), jnp.float32)
        for r in range(16):
            for cc in range(4):
                part_v[r, pl.ds(cc * 16, 16)] = zf
        for kk in range(3):
            @pl.when(slot_local == kk)
            def _(kk=kk):
                for cc in range(4):
                    part_v[kk, pl.ds(cc * 16, 16)] = accs[cc]
        pltpu.sync_copy(rows16, widx_v)
        pltpu.sync_copy(part_v, shared.at[widx_v], add=True)

    plsc.subcore_barrier()

    @pl.when(si == 15)
    def _():
        pltpu.sync_copy(shared, part_v)
        scale = jnp.float32(1.0 / _L)
        for kk in range(3):
            for cc in range(4):
                outv_v[pl.ds(kk * 64 + cc * 16, 16)] = (
                    part_v[kk, pl.ds(cc * 16, 16)] * scale)
        for cj in range(2):
            @pl.when(c == cj)
            def _(cj=cj):
                pltpu.sync_copy(outv_v, out.at[pl.ds(cj * 192, 192)])


@jax.jit
def _run(idx_list, tab_list):
    kfn = pl.kernel(
        _sc_body,
        out_type=jax.ShapeDtypeStruct((_NSLOT * _D,), jnp.float32),
        mesh=plsc.VectorSubcoreMesh(core_axis_name="c", subcore_axis_name="s"),
        compiler_params=pltpu.CompilerParams(needs_layout_passes=False),
        scratch_types=[
            pltpu.VMEM((256,), jnp.int32),
            pltpu.VMEM((3, _B, 8, 8, 128), jnp.float32),
            pltpu.VMEM((16, _D), jnp.float32),
            pltpu.VMEM((192,), jnp.float32),
            pltpu.VMEM((16,), jnp.int32),
            pltpu.VMEM_SHARED((16, _D), jnp.float32),
            pltpu.SemaphoreType.DMA,
            pltpu.SemaphoreType.DMA,
            pltpu.SemaphoreType.DMA,
        ],
    )
    rows16 = jnp.asarray(np.arange(16, dtype=np.int32))
    return kfn(*idx_list, rows16, *tab_list)


def kernel(scene, subject, action, object, purpose, result,
           W_scene, W_subject, W_action, W_object, W_purpose, W_result):
    idx_list = [x.reshape(_L).astype(jnp.int32)
                for x in (scene, subject, action, object, purpose, result)]
    tab_list = [w.T.reshape(8, 8, _V)
                for w in (W_scene, W_subject, W_action, W_object,
                          W_purpose, W_result)]
    return _run(idx_list, tab_list)


# speedup vs baseline: 1.0697x; 1.0052x over previous
import numpy as np

import jax
import jax.numpy as jnp
from jax import lax
from jax.experimental import pallas as pl
from jax.experimental.pallas import tpu as pltpu
from jax.experimental.pallas import tpu_sc as plsc

_L = 200
_D = 64
_NSLOT = 6
_TPS = 5
_NI = _L // _TPS
_B = 5
_NB = _NI // _B
_V = 1000000


def _extract(iota, iv, ln, zero16):
    return jnp.max(jnp.where(iota == jnp.int32(ln), iv, zero16))


def _sc_body(i0, i1, i2, i3, i4, i5, rows16, t0, t1, t2, t3, t4, t5, out,
             idx_v, rows_v, part_v, outv_v, widx_v, shared, sem0, sem1, sem2):
    c = lax.axis_index("c")
    si = lax.axis_index("s")
    idxs = (i0, i1, i2, i3, i4, i5)
    tabs = (t0, t1, t2, t3, t4, t5)
    sems = (sem0, sem1, sem2)
    slot_local = si // _TPS
    t = si % _TPS
    is_worker = si < 15
    iota = lax.iota(jnp.int32, 16)
    zero16 = jnp.zeros((16,), jnp.int32)
    base = _NI * t

    @pl.when(si == 15)
    def _():
        zf = jnp.zeros((16,), jnp.float32)
        for r in range(16):
            for cc in range(4):
                part_v[r, pl.ds(cc * 16, 16)] = zf
        pltpu.sync_copy(part_v, shared)

    plsc.subcore_barrier()

    for k in range(_NSLOT):
        @pl.when(jnp.logical_and(is_worker, (c * 3 + slot_local) == k))
        def _(k=k):
            pltpu.sync_copy(idxs[k], idx_v.at[pl.ds(0, _L)])

    def fire(w):
        for k in range(_NSLOT):
            @pl.when(jnp.logical_and(is_worker, (c * 3 + slot_local) == k))
            def _(k=k, w=w):
                for j in range(_B):
                    n = w * _B + j
                    g, ln = n // 16, n % 16
                    iv = idx_v[pl.ds(base + g * 16, 16)]
                    s = _extract(iota, iv, ln, zero16)
                    s128 = pl.multiple_of((s >> 7) << 7, 128)
                    pltpu.async_copy(
                        tabs[k].at[:, :, pl.ds(s128, 128)],
                        rows_v.at[w % 3, j], sems[w % 3])

    avecs = [(iota + jnp.int32(cc * 16)) >> 3 for cc in range(4)]
    bvecs = [(iota + jnp.int32(cc * 16)) & jnp.int32(7) for cc in range(4)]

    def drain_acc(w, accs):
        @pl.when(is_worker)
        def _():
            for j in range(_B):
                pltpu.make_async_copy(
                    tabs[0].at[:, :, pl.ds(0, 128)],
                    rows_v.at[w % 3, j], sems[w % 3]).wait()
        new = []
        for cc in range(4):
            acc = accs[cc]
            for j in range(_B):
                n = w * _B + j
                g, ln = n // 16, n % 16
                iv = idx_v[pl.ds(base + g * 16, 16)]
                rm = _extract(iota, iv & jnp.int32(127), ln, zero16)
                rv = jnp.full((16,), rm, jnp.int32)
                jv = jnp.full((16,), j, jnp.int32)
                bufv = jnp.full((16,), w % 3, jnp.int32)
                vals = plsc.load_gather(
                    rows_v, [bufv, jv, avecs[cc], bvecs[cc], rv])
                acc = jnp.where(is_worker, acc + vals, acc)
            new.append(acc)
        return new

    accs = [jnp.zeros((16,), jnp.float32) for _ in range(4)]
    fire(0)
    fire(1)
    for w in range(2, _NB):
        fire(w)
        accs = drain_acc(w - 2, accs)
    accs = drain_acc(_NB - 2, accs)
    accs = drain_acc(_NB - 1, accs)

    @pl.when(is_worker)
    def _():
        zf = jnp.zeros((16,), jnp.float32)
        for r in range(16):
            for cc in range(4):
                part_v[r, pl.ds(cc * 16, 16)] = zf
        for kk in range(3):
            @pl.when(slot_local == kk)
            def _(kk=kk):
                for cc in range(4):
                    part_v[kk, pl.ds(cc * 16, 16)] = accs[cc]
        pltpu.sync_copy(rows16, widx_v)
        pltpu.sync_copy(part_v, shared.at[widx_v], add=True)

    plsc.subcore_barrier()

    @pl.when(si == 15)
    def _():
        pltpu.sync_copy(shared, part_v)
        scale = jnp.float32(1.0 / _L)
        for kk in range(3):
            for cc in range(4):
                outv_v[pl.ds(kk * 64 + cc * 16, 16)] = (
                    part_v[kk, pl.ds(cc * 16, 16)] * scale)
        for cj in range(2):
            @pl.when(c == cj)
            def _(cj=cj):
                pltpu.sync_copy(outv_v, out.at[pl.ds(cj * 192, 192)])


@jax.jit
def _run(idx_list, tab_list):
    kfn = pl.kernel(
        _sc_body,
        out_type=jax.ShapeDtypeStruct((_NSLOT * _D,), jnp.float32),
        mesh=plsc.VectorSubcoreMesh(core_axis_name="c", subcore_axis_name="s"),
        compiler_params=pltpu.CompilerParams(needs_layout_passes=False),
        scratch_types=[
            pltpu.VMEM((256,), jnp.int32),
            pltpu.VMEM((3, _B, 8, 8, 128), jnp.float32),
            pltpu.VMEM((16, _D), jnp.float32),
            pltpu.VMEM((192,), jnp.float32),
            pltpu.VMEM((16,), jnp.int32),
            pltpu.VMEM_SHARED((16, _D), jnp.float32),
            pltpu.SemaphoreType.DMA,
            pltpu.SemaphoreType.DMA,
            pltpu.SemaphoreType.DMA,
        ],
    )
    rows16 = jnp.asarray(np.arange(16, dtype=np.int32))
    return kfn(*idx_list, rows16, *tab_list)


def kernel(scene, subject, action, object, purpose, result,
           W_scene, W_subject, W_action, W_object, W_purpose, W_result):
    idx_list = [x.reshape(_L).astype(jnp.int32)
                for x in (scene, subject, action, object, purpose, result)]
    tab_list = [w.T.reshape(8, 8, _V)
                for w in (W_scene, W_subject, W_action, W_object,
                          W_purpose, W_result)]
    return _run(idx_list, tab_list)
